# baseline (speedup 1.0000x reference)
import functools

import jax
import jax.numpy as jnp
from jax import lax
from jax.experimental import pallas as pl
from jax.experimental.pallas import tpu as pltpu
from jax.experimental.pallas import tpu_sc as plsc

_VOCAB = 100000
_D = 64
_B = 4096
_L = 200

_NC = 2
_NS = 16
_NW = _NC * _NS
_BPW = _B // _NW
_CHUNK = 100
_NCHUNK = _L // _CHUNK
_ROWS = _BPW * _NCHUNK

_VB = 1024
_NV = (_VOCAB + _VB - 1) // _VB


def _accum4(rows_ref, acc):
    def body(t, a):
        return tuple(a[q] + rows_ref[t, pl.ds(16 * q, 16)] for q in range(4))
    return plsc.parallel_loop(0, _CHUNK, unroll=10, carry=acc)(body)


_NBUF = 8
_NHALF = 2
_BH = _B // _NHALF
_BPWH = _BH // _NW
_ROWSH = _BPWH * _NCHUNK


def _pool_body(half, ids_hbm, table_hbm, out_hbm, idx_v, *rest):
    rows = rest[:_NBUF]
    out_v = rest[_NBUF]
    sems = rest[_NBUF + 1:]
    c = lax.axis_index("c")
    s = lax.axis_index("s")
    wid = s * _NC + c
    pltpu.sync_copy(
        ids_hbm.at[pl.ds(half * _NW * _ROWSH + wid * _ROWSH, _ROWSH)], idx_v)
    for q in range(_NBUF):
        pltpu.async_copy(table_hbm.at[idx_v.at[q]], rows[q], sems[q])

    def group_body(g, carry):
        k0 = _NBUF * g
        for p in range(_NBUF // 2):
            zero = jnp.zeros((16,), jnp.float32)
            acc = (zero, zero, zero, zero)
            for q in (2 * p, 2 * p + 1):
                pltpu.make_async_copy(
                    table_hbm.at[idx_v.at[k0 + q]], rows[q], sems[q]).wait()
                acc = _accum4(rows[q], acc)

                @pl.when(k0 + _NBUF + q < _ROWSH)
                def _start_next():
                    pltpu.async_copy(
                        table_hbm.at[idx_v.at[k0 + _NBUF + q]],
                        rows[q], sems[q])

            base = (_NBUF // 2 * g + p) * _D
            for q in range(4):
                out_v[pl.ds(base + 16 * q, 16)] = acc[q]
        return carry

    lax.fori_loop(0, _ROWSH // _NBUF, group_body, 0)
    pltpu.sync_copy(out_v, out_hbm.at[pl.ds(wid * _BPWH * _D, _BPWH * _D)])


@functools.cache
def _make_pool(half):
    return pl.kernel(
        functools.partial(_pool_body, half),
        out_type=jax.ShapeDtypeStruct((_BH * _D,), jnp.float32),
        mesh=plsc.VectorSubcoreMesh(core_axis_name="c", subcore_axis_name="s"),
        scratch_types=(
            [pltpu.VMEM((_ROWSH, _CHUNK), jnp.int32)]
            + [pltpu.VMEM((_CHUNK, _D), jnp.float32)] * _NBUF
            + [pltpu.VMEM((_BPWH * _D,), jnp.float32)]
            + [pltpu.SemaphoreType.DMA] * _NBUF
        ),
        compiler_params=pltpu.CompilerParams(use_tc_tiling_on_sc=False),
    )


def _head_body(pooled_ref, gamma_ref, beta_ref, W1_ref, b1_ref, Wv1_ref,
               bv1_ref, Wv2_ref, bv2_ref, W2T_ref, b2T_ref, *rest):
    if len(rest) == 4:
        rest = rest[1:]
    logitsT_ref, value_ref, ht_scr = rest
    v = pl.program_id(0)

    @pl.when(v == 0)
    def _small_stage():
        x = pooled_ref[...] * (1.0 / _L)
        mu = jnp.mean(x, axis=-1, keepdims=True)
        xc = x - mu
        var = jnp.mean(xc * xc, axis=-1, keepdims=True)
        xn = xc * lax.rsqrt(var + 1e-5) * gamma_ref[...] + beta_ref[...]
        h = jnp.tanh(
            jnp.dot(xn, W1_ref[...], preferred_element_type=jnp.float32)
            + b1_ref[...])
        ht_scr[...] = h.T
        hv = jnp.tanh(
            jnp.dot(xn, Wv1_ref[...], preferred_element_type=jnp.float32)
            + bv1_ref[...])
        value_ref[...] = (
            jnp.dot(hv, Wv2_ref[...], preferred_element_type=jnp.float32)
            + bv2_ref[...])

    b2col = jnp.swapaxes(b2T_ref[...], 0, 1)
    logitsT_ref[...] = (
        lax.dot_general(W2T_ref[...], ht_scr[...],
                        (((0,), (0,)), ((), ())),
                        preferred_element_type=jnp.float32)
        + b2col)


@functools.cache
def _make_heads(half, aliased):
    full = lambda shape: pl.BlockSpec(shape, lambda v: (0,) * len(shape))
    in_specs = [
        full((_BH, _D)),
        full((1, _D)),
        full((1, _D)),
        full((_D, _D)),
        full((1, _D)),
        full((_D, _D)),
        full((1, _D)),
        full((_D, 1)),
        full((1, 1)),
        pl.BlockSpec((_D, _VB), lambda v: (0, v)),
        pl.BlockSpec((1, _VB), lambda v: (0, v)),
    ]
    aliases = {}
    if aliased:
        in_specs.append(pl.BlockSpec(memory_space=pl.ANY))
        aliases = {11: 0}
    return pl.pallas_call(
        _head_body,
        grid=(_NV,),
        in_specs=in_specs,
        out_specs=[
            pl.BlockSpec((_VB, _BH), lambda v, h=half: (v, h)),
            pl.BlockSpec((_BH, 1), lambda v: (0, 0)),
        ],
        out_shape=[
            jax.ShapeDtypeStruct((_VOCAB, _B), jnp.float32),
            jax.ShapeDtypeStruct((_BH, 1), jnp.float32),
        ],
        scratch_shapes=[pltpu.VMEM((_D, _BH), jnp.float32)],
        input_output_aliases=aliases,
    )


def kernel(obs_ids, table, gamma, beta, W1, b1, W2, b2, Wv1, bv1, Wv2, bv2):
    ids2 = obs_ids.reshape(_B * _NCHUNK, _CHUNK).astype(jnp.int32)
    weights = (
        gamma.reshape(1, _D), beta.reshape(1, _D),
        W1, b1.reshape(1, _D),
        Wv1, bv1.reshape(1, _D),
        Wv2, bv2.reshape(1, 1),
        W2, b2.reshape(1, _VOCAB),
    )
    pooled0 = _make_pool(0)(ids2, table).reshape(_BH, _D)
    logitsT, v0 = _make_heads(0, False)(pooled0, *weights)
    pooled1 = _make_pool(1)(ids2, table).reshape(_BH, _D)
    logitsT, v1 = _make_heads(1, True)(pooled1, *weights, logitsT)
    value = jnp.concatenate([v0, v1], axis=0).reshape(_B)
    return logitsT.T, value


# speedup vs baseline: 1.0385x; 1.0385x over previous
import functools

import jax
import jax.numpy as jnp
from jax import lax
from jax.experimental import pallas as pl
from jax.experimental.pallas import tpu as pltpu
from jax.experimental.pallas import tpu_sc as plsc

_VOCAB = 100000
_D = 64
_B = 4096
_L = 200

_NC = 2
_NS = 16
_NW = _NC * _NS
_BPW = _B // _NW
_CHUNK = 100
_NCHUNK = _L // _CHUNK
_ROWS = _BPW * _NCHUNK

_VB = 1024
_NV = (_VOCAB + _VB - 1) // _VB


def _accum4(rows_ref, acc):
    def body(t, a):
        return tuple(a[q] + rows_ref[t, pl.ds(16 * q, 16)] for q in range(4))
    return plsc.parallel_loop(0, _CHUNK, unroll=10, carry=acc)(body)


_NBUF = 8


def _pool_body(ids_hbm, table_hbm, out_hbm, idx_v, *rest):
    rows = rest[:_NBUF]
    out_v = rest[_NBUF]
    sems = rest[_NBUF + 1:]
    c = lax.axis_index("c")
    s = lax.axis_index("s")
    wid = s * _NC + c
    pltpu.sync_copy(ids_hbm.at[pl.ds(wid * _ROWS, _ROWS)], idx_v)
    for q in range(_NBUF):
        pltpu.async_copy(table_hbm.at[idx_v.at[q]], rows[q], sems[q])

    def group_body(g, carry):
        k0 = _NBUF * g
        for half in range(_NBUF // 2):
            zero = jnp.zeros((16,), jnp.float32)
            acc = (zero, zero, zero, zero)
            for q in (2 * half, 2 * half + 1):
                pltpu.make_async_copy(
                    table_hbm.at[idx_v.at[k0 + q]], rows[q], sems[q]).wait()
                acc = _accum4(rows[q], acc)

                @pl.when(k0 + _NBUF + q < _ROWS)
                def _start_next():
                    pltpu.async_copy(
                        table_hbm.at[idx_v.at[k0 + _NBUF + q]],
                        rows[q], sems[q])

            base = (_NBUF // 2 * g + half) * _D
            for q in range(4):
                out_v[pl.ds(base + 16 * q, 16)] = acc[q]
        return carry

    lax.fori_loop(0, _ROWS // _NBUF, group_body, 0)
    pltpu.sync_copy(out_v, out_hbm.at[pl.ds(wid * _BPW * _D, _BPW * _D)])


@functools.cache
def _make_pool():
    return pl.kernel(
        _pool_body,
        out_type=jax.ShapeDtypeStruct((_B * _D,), jnp.float32),
        mesh=plsc.VectorSubcoreMesh(core_axis_name="c", subcore_axis_name="s"),
        scratch_types=(
            [pltpu.VMEM((_ROWS, _CHUNK), jnp.int32)]
            + [pltpu.VMEM((_CHUNK, _D), jnp.float32)] * _NBUF
            + [pltpu.VMEM((_BPW * _D,), jnp.float32)]
            + [pltpu.SemaphoreType.DMA] * _NBUF
        ),
        compiler_params=pltpu.CompilerParams(use_tc_tiling_on_sc=False),
    )


def _head_body(pooled_ref, gamma_ref, beta_ref, W1_ref, b1_ref, Wv1_ref,
               bv1_ref, Wv2_ref, bv2_ref, W2T_ref, b2T_ref,
               logitsT_ref, value_ref, ht_scr):
    v = pl.program_id(0)

    @pl.when(v == 0)
    def _small_stage():
        x = pooled_ref[...] * (1.0 / _L)
        mu = jnp.mean(x, axis=-1, keepdims=True)
        xc = x - mu
        var = jnp.mean(xc * xc, axis=-1, keepdims=True)
        xn = xc * lax.rsqrt(var + 1e-5) * gamma_ref[...] + beta_ref[...]
        h = jnp.tanh(
            jnp.dot(xn, W1_ref[...], preferred_element_type=jnp.float32)
            + b1_ref[...])
        ht_scr[...] = h.T
        hv = jnp.tanh(
            jnp.dot(xn, Wv1_ref[...], preferred_element_type=jnp.float32)
            + bv1_ref[...])
        value_ref[...] = (
            jnp.dot(hv, Wv2_ref[...], preferred_element_type=jnp.float32)
            + bv2_ref[...])

    b2col = jnp.swapaxes(b2T_ref[...], 0, 1)
    logitsT_ref[...] = (
        lax.dot_general(W2T_ref[...], ht_scr[...],
                        (((0,), (0,)), ((), ())),
                        preferred_element_type=jnp.float32)
        + b2col)


@functools.cache
def _make_heads():
    full = lambda shape: pl.BlockSpec(shape, lambda v: (0,) * len(shape))
    return pl.pallas_call(
        _head_body,
        grid=(_NV,),
        in_specs=[
            full((_B, _D)),
            full((1, _D)),
            full((1, _D)),
            full((_D, _D)),
            full((1, _D)),
            full((_D, _D)),
            full((1, _D)),
            full((_D, 1)),
            full((1, 1)),
            pl.BlockSpec((_D, _VB), lambda v: (0, v)),
            pl.BlockSpec((1, _VB), lambda v: (0, v)),
        ],
        out_specs=[
            pl.BlockSpec((_VB, _B), lambda v: (v, 0)),
            pl.BlockSpec((_B, 1), lambda v: (0, 0)),
        ],
        out_shape=[
            jax.ShapeDtypeStruct((_VOCAB, _B), jnp.float32),
            jax.ShapeDtypeStruct((_B, 1), jnp.float32),
        ],
        scratch_shapes=[pltpu.VMEM((_D, _B), jnp.float32)],
    )


def kernel(obs_ids, table, gamma, beta, W1, b1, W2, b2, Wv1, bv1, Wv2, bv2):
    ids2 = obs_ids.reshape(_B * _NCHUNK, _CHUNK).astype(jnp.int32)
    pooled = _make_pool()(ids2, table).reshape(_B, _D)
    logitsT, value = _make_heads()(
        pooled,
        gamma.reshape(1, _D), beta.reshape(1, _D),
        W1, b1.reshape(1, _D),
        Wv1, bv1.reshape(1, _D),
        Wv2, bv2.reshape(1, 1),
        W2, b2.reshape(1, _VOCAB),
    )
    return logitsT.T, value.reshape(_B)
